# baseline (speedup 1.0000x reference)
import functools
import jax
import jax.numpy as jnp
from jax import lax
from jax.experimental import pallas as pl
from jax.experimental.pallas import tpu as pltpu
from jax.experimental.pallas import tpu_sc as plsc

H = 8
D = 16
OUT = 128
IN = 128
N = 10000
E = 320000

NC = 2
NS = 16
LANES = 16

C = 40
EPW = E // NS
CHUNKS = EPW // C
NPAD = 10240
ROWS_PER_TILE = NPAD // NS


def _pre_body(x_ref, w_ref, b_ref, q_ref, kv_ref):
    y = jnp.dot(x_ref[...], w_ref[...], preferred_element_type=jnp.float32)
    y = y + b_ref[...]
    q_ref[...] = y[:, 0:OUT]
    kv_ref[...] = y[:, OUT:3 * OUT]


def _tc_pre(x, w, b):
    BN = 1000
    grid = (N // BN,)
    return pl.pallas_call(
        _pre_body,
        grid=grid,
        in_specs=[
            pl.BlockSpec((BN, IN), lambda i: (i, 0)),
            pl.BlockSpec((IN, 3 * OUT), lambda i: (0, 0)),
            pl.BlockSpec((1, 3 * OUT), lambda i: (0, 0)),
        ],
        out_specs=[
            pl.BlockSpec((BN, OUT), lambda i: (i, 0)),
            pl.BlockSpec((BN, 2 * OUT), lambda i: (i, 0)),
        ],
        out_shape=[
            jax.ShapeDtypeStruct((N, OUT), jnp.float32),
            jax.ShapeDtypeStruct((N, 2 * OUT), jnp.float32),
        ],
    )(x, w, b)


def _post_body(agg_ref, s_ref, x_ref, w_ref, b_ref, al_ref, g_ref, be_ref,
               o_ref):
    agg = agg_ref[...]
    den = s_ref[...][:, 0:H] + 1e-16
    bn = agg.shape[0]
    aggn = (agg.reshape(bn, H, D) / den[:, :, None]).reshape(bn, OUT)
    o = jnp.dot(jax.nn.gelu(aggn), w_ref[...],
                preferred_element_type=jnp.float32) + b_ref[...]
    al = al_ref[...]
    r = al * o + (1.0 - al) * x_ref[...]
    r = jnp.maximum(r, 0.0)
    mu = jnp.mean(r, axis=-1, keepdims=True)
    var = jnp.mean((r - mu) ** 2, axis=-1, keepdims=True)
    rn = (r - mu) * lax.rsqrt(var + 1e-5)
    o_ref[...] = g_ref[...] * rn + be_ref[...]


def _tc_post(agg, s, x, w, b, al, gamma, beta):
    BN = 1000
    grid = (N // BN,)
    return pl.pallas_call(
        _post_body,
        grid=grid,
        in_specs=[
            pl.BlockSpec((BN, OUT), lambda i: (i, 0)),
            pl.BlockSpec((BN, 2 * H), lambda i: (i, 0)),
            pl.BlockSpec((BN, IN), lambda i: (i, 0)),
            pl.BlockSpec((OUT, OUT), lambda i: (0, 0)),
            pl.BlockSpec((1, OUT), lambda i: (0, 0)),
            pl.BlockSpec((1, 1), lambda i: (0, 0)),
            pl.BlockSpec((1, OUT), lambda i: (0, 0)),
            pl.BlockSpec((1, OUT), lambda i: (0, 0)),
        ],
        out_specs=pl.BlockSpec((BN, OUT), lambda i: (i, 0)),
        out_shape=jax.ShapeDtypeStruct((N, OUT), jnp.float32),
    )(agg, s, x, w, b, al, gamma, beta)


def _edge_chunks(tid, q_hbm, kv_hbm, src_hbm, dst_hbm,
                 idxs, idxd, qb, kvb, mb, wb, agg_sh, s_sh, sem):
    def chunk(c, _):
        base = tid * EPW + c * C
        pltpu.sync_copy(src_hbm.at[pl.ds(base, C)], idxs)
        pltpu.sync_copy(dst_hbm.at[pl.ds(base, C)], idxd)
        cp_q = pltpu.async_copy(q_hbm.at[idxd], qb, sem)
        cp_kv = pltpu.async_copy(kv_hbm.at[idxs], kvb, sem)
        cp_q.wait()
        cp_kv.wait()

        lanes = lax.iota(jnp.int32, LANES)

        def edge(e, _):
            wvec = jnp.zeros((LANES,), jnp.float32)
            for h in range(H):
                prod = qb[e, pl.ds(h * D, D)] * kvb[e, pl.ds(h * D, D)]
                sh = jnp.sum(prod)
                wsp = jnp.exp(jnp.full((LANES,), sh, jnp.float32))
                mb[e, pl.ds(h * D, D)] = kvb[e, pl.ds(OUT + h * D, D)] * wsp
                wvec = jnp.where(lanes == h, wsp, wvec)
            wb[e, pl.ds(0, LANES)] = wvec
            return 0

        lax.fori_loop(0, C, edge, 0)
        pltpu.sync_copy(mb, agg_sh.at[idxd], add=True)
        pltpu.sync_copy(wb, s_sh.at[idxd], add=True)
        return 0

    lax.fori_loop(0, CHUNKS, chunk, 0)


def _sc_body(q0, kv0, q1, kv1, s0, d0, s1, d1,
             agg0, agg1, den0, den1,
             idxs, idxd, qb, kvb, mb, wb, agg_sh, s_sh, sem):
    cid = lax.axis_index("c")
    tid = lax.axis_index("s")

    zv = jnp.zeros((LANES,), jnp.float32)

    def zero_mb(i, _):
        r = i // (OUT // LANES)
        cc = i % (OUT // LANES)
        mb[r, pl.ds(cc * LANES, LANES)] = zv
        return 0

    lax.fori_loop(0, C * (OUT // LANES), zero_mb, 0)

    def zero_wb(i, _):
        wb[i, pl.ds(0, LANES)] = zv
        return 0

    lax.fori_loop(0, C, zero_wb, 0)

    def zero_stripe(j, _):
        r0z = tid * ROWS_PER_TILE + j * C
        pltpu.sync_copy(mb, agg_sh.at[pl.ds(r0z, C)])
        pltpu.sync_copy(wb, s_sh.at[pl.ds(r0z, C)])
        return 0

    lax.fori_loop(0, ROWS_PER_TILE // C, zero_stripe, 0)

    plsc.subcore_barrier()

    @pl.when(cid == 0)
    def _():
        _edge_chunks(tid, q0, kv0, s0, d0,
                     idxs, idxd, qb, kvb, mb, wb, agg_sh, s_sh, sem)

    @pl.when(cid == 1)
    def _():
        _edge_chunks(tid, q1, kv1, s1, d1,
                     idxs, idxd, qb, kvb, mb, wb, agg_sh, s_sh, sem)

    plsc.subcore_barrier()

    r0 = tid * ROWS_PER_TILE

    @pl.when(cid == 0)
    def _():
        pltpu.sync_copy(agg_sh.at[pl.ds(r0, ROWS_PER_TILE)],
                        agg0.at[pl.ds(r0, ROWS_PER_TILE)])
        pltpu.sync_copy(s_sh.at[pl.ds(r0, ROWS_PER_TILE)],
                        den0.at[pl.ds(r0, ROWS_PER_TILE)])

    @pl.when(cid == 1)
    def _():
        pltpu.sync_copy(agg_sh.at[pl.ds(r0, ROWS_PER_TILE)],
                        agg1.at[pl.ds(r0, ROWS_PER_TILE)])
        pltpu.sync_copy(s_sh.at[pl.ds(r0, ROWS_PER_TILE)],
                        den1.at[pl.ds(r0, ROWS_PER_TILE)])


_sc_edge = functools.partial(
    pl.kernel,
    out_type=[
        jax.ShapeDtypeStruct((NPAD, OUT), jnp.float32),
        jax.ShapeDtypeStruct((NPAD, OUT), jnp.float32),
        jax.ShapeDtypeStruct((NPAD, 2 * H), jnp.float32),
        jax.ShapeDtypeStruct((NPAD, 2 * H), jnp.float32),
    ],
    mesh=plsc.VectorSubcoreMesh(core_axis_name="c", subcore_axis_name="s",
                                num_cores=NC, num_subcores=NS),
    compiler_params=pltpu.CompilerParams(needs_layout_passes=False,
                                         use_tc_tiling_on_sc=False),
    scratch_types=[
        pltpu.VMEM((C,), jnp.int32),
        pltpu.VMEM((C,), jnp.int32),
        pltpu.VMEM((C, OUT), jnp.float32),
        pltpu.VMEM((C, 2 * OUT), jnp.float32),
        pltpu.VMEM((C, OUT), jnp.float32),
        pltpu.VMEM((C, 2 * H), jnp.float32),
        pltpu.VMEM_SHARED((NPAD, OUT), jnp.float32),
        pltpu.VMEM_SHARED((NPAD, 2 * H), jnp.float32),
        pltpu.SemaphoreType.DMA,
    ],
)(_sc_body)


def _fold_weights(W_kqv, b_kqv, p_q, a_k, m_v):
    Wk, Wq, Wv = jnp.split(W_kqv, 3, axis=1)
    bk, bq, bv = jnp.split(b_kqv, 3)
    sc = (p_q / jnp.sqrt(jnp.float32(D)))
    Wq2 = (Wq.reshape(IN, H, D) * sc[None, :, None]).reshape(IN, OUT)
    bq2 = (bq.reshape(H, D) * sc[:, None]).reshape(OUT)
    Wk2 = jnp.einsum('ihd,hde->ihe', Wk.reshape(IN, H, D), a_k).reshape(IN, OUT)
    bk2 = jnp.einsum('hd,hde->he', bk.reshape(H, D), a_k).reshape(OUT)
    Wv2 = jnp.einsum('ihd,hde->ihe', Wv.reshape(IN, H, D), m_v).reshape(IN, OUT)
    bv2 = jnp.einsum('hd,hde->he', bv.reshape(H, D), m_v).reshape(OUT)
    W = jnp.concatenate([Wq2, Wk2, Wv2], axis=1)
    b = jnp.concatenate([bq2, bk2, bv2])[None, :]
    return W, b


def kernel(x_paper, x_author, edge_index_writes, edge_index_rev,
           W_kqv_paper, b_kqv_paper, W_kqv_author, b_kqv_author,
           a_writes, m_writes, p_writes, a_rev, m_rev, p_rev,
           W_out_paper, b_out_paper, W_out_author, b_out_author,
           skip_paper, skip_author, ln_gamma, ln_beta):
    W_p, b_p = _fold_weights(W_kqv_paper, b_kqv_paper, p_writes, a_rev, m_rev)
    W_a, b_a = _fold_weights(W_kqv_author, b_kqv_author, p_rev,
                             a_writes, m_writes)

    q_p, kv_p = _tc_pre(x_paper, W_p, b_p)
    q_a, kv_a = _tc_pre(x_author, W_a, b_a)

    agg_p, agg_a, den_p, den_a = _sc_edge(
        q_p, kv_a, q_a, kv_p,
        edge_index_writes[0], edge_index_writes[1],
        edge_index_rev[0], edge_index_rev[1])

    al_p = jax.nn.sigmoid(skip_paper).reshape(1, 1)
    al_a = jax.nn.sigmoid(skip_author).reshape(1, 1)
    gamma = ln_gamma[None, :]
    beta = ln_beta[None, :]
    b_out_p = b_out_paper[None, :]
    b_out_a = b_out_author[None, :]

    out_p = _tc_post(agg_p, den_p, x_paper, W_out_paper, b_out_p,
                     al_p, gamma, beta)
    out_a = _tc_post(agg_a, den_a, x_author, W_out_author,
                     b_out_a, al_a, gamma, beta)
    return out_p, out_a


# speedup vs baseline: 3.0484x; 3.0484x over previous
import functools
import jax
import jax.numpy as jnp
from jax import lax
from jax.experimental import pallas as pl
from jax.experimental.pallas import tpu as pltpu
from jax.experimental.pallas import tpu_sc as plsc

H = 8
D = 16
OUT = 128
IN = 128
N = 10000
E = 320000

NC = 2
NS = 16
LANES = 16

C = 32
EPW = E // NS
CHUNKS = EPW // C
NPAD = 10240
ROWS_PER_TILE = NPAD // NS


def _pre_body(x_ref, w_ref, b_ref, q_ref, k_ref, v_ref):
    y = jnp.dot(x_ref[...], w_ref[...], preferred_element_type=jnp.float32)
    y = y + b_ref[...]
    q_ref[...] = y[:, 0:OUT]
    k_ref[...] = y[:, OUT:2 * OUT]
    v_ref[...] = y[:, 2 * OUT:3 * OUT]


def _tc_pre(x, w, b):
    BN = 1000
    grid = (N // BN,)
    return pl.pallas_call(
        _pre_body,
        grid=grid,
        in_specs=[
            pl.BlockSpec((BN, IN), lambda i: (i, 0)),
            pl.BlockSpec((IN, 3 * OUT), lambda i: (0, 0)),
            pl.BlockSpec((1, 3 * OUT), lambda i: (0, 0)),
        ],
        out_specs=[
            pl.BlockSpec((BN, OUT), lambda i: (i, 0)),
            pl.BlockSpec((BN, OUT), lambda i: (i, 0)),
            pl.BlockSpec((BN, OUT), lambda i: (i, 0)),
        ],
        out_shape=[jax.ShapeDtypeStruct((N, OUT), jnp.float32)] * 3,
    )(x, w, b)


def _post_body(agg_ref, s_ref, x_ref, w_ref, b_ref, al_ref, g_ref, be_ref,
               o_ref):
    agg = agg_ref[...]
    den = s_ref[...][:, 0:H] + 1e-16
    bn = agg.shape[0]
    aggn = (agg.reshape(bn, H, D) / den[:, :, None]).reshape(bn, OUT)
    o = jnp.dot(jax.nn.gelu(aggn), w_ref[...],
                preferred_element_type=jnp.float32) + b_ref[...]
    al = al_ref[...]
    r = al * o + (1.0 - al) * x_ref[...]
    r = jnp.maximum(r, 0.0)
    mu = jnp.mean(r, axis=-1, keepdims=True)
    var = jnp.mean((r - mu) ** 2, axis=-1, keepdims=True)
    rn = (r - mu) * lax.rsqrt(var + 1e-5)
    o_ref[...] = g_ref[...] * rn + be_ref[...]


def _tc_post(agg, s, x, w, b, al, gamma, beta):
    BN = 1000
    grid = (N // BN,)
    return pl.pallas_call(
        _post_body,
        grid=grid,
        in_specs=[
            pl.BlockSpec((BN, OUT), lambda i: (i, 0)),
            pl.BlockSpec((BN, 2 * H), lambda i: (i, 0)),
            pl.BlockSpec((BN, IN), lambda i: (i, 0)),
            pl.BlockSpec((OUT, OUT), lambda i: (0, 0)),
            pl.BlockSpec((1, OUT), lambda i: (0, 0)),
            pl.BlockSpec((1, 1), lambda i: (0, 0)),
            pl.BlockSpec((1, OUT), lambda i: (0, 0)),
            pl.BlockSpec((1, OUT), lambda i: (0, 0)),
        ],
        out_specs=pl.BlockSpec((BN, OUT), lambda i: (i, 0)),
        out_shape=jax.ShapeDtypeStruct((N, OUT), jnp.float32),
    )(agg, s, x, w, b, al, gamma, beta)


def _edge_chunks(tid, q_hbm, k_hbm, v_hbm, src_hbm, dst_hbm,
                 idxs0, idxd0, idxs1, idxd1,
                 qb0, kb0, vb0, qb1, kb1, vb1,
                 mb, wb, agg_sh, s_sh, sem_a, sem_b):
    base0 = tid * EPW
    lanes = lax.iota(jnp.int32, LANES)

    def compute_scatter(qb, kb, vb, idxd):
        def edge(e, _):
            wvec = jnp.zeros((LANES,), jnp.float32)
            for h in range(H):
                prod = qb[e, pl.ds(h * D, D)] * kb[e, pl.ds(h * D, D)]
                sh = jnp.sum(prod)
                wsp = jnp.exp(jnp.full((LANES,), sh, jnp.float32))
                mb[e, pl.ds(h * D, D)] = vb[e, pl.ds(h * D, D)] * wsp
                wvec = jnp.where(lanes == h, wsp, wvec)
            wb[e, pl.ds(0, LANES)] = wvec
            return 0

        lax.fori_loop(0, C, edge, 0)
        pltpu.sync_copy(mb, agg_sh.at[idxd], add=True)
        pltpu.sync_copy(wb, s_sh.at[idxd], add=True)

    def fetch(base, idxs, idxd, qb, kb, vb, sem):
        pltpu.sync_copy(src_hbm.at[pl.ds(base, C)], idxs)
        pltpu.sync_copy(dst_hbm.at[pl.ds(base, C)], idxd)
        pltpu.async_copy(q_hbm.at[idxd], qb, sem)
        pltpu.async_copy(k_hbm.at[idxs], kb, sem)
        pltpu.async_copy(v_hbm.at[idxs], vb, sem)

    def drain(idxs, idxd, qb, kb, vb, sem):
        pltpu.make_async_copy(q_hbm.at[idxd], qb, sem).wait()
        pltpu.make_async_copy(k_hbm.at[idxs], kb, sem).wait()
        pltpu.make_async_copy(v_hbm.at[idxs], vb, sem).wait()

    fetch(base0, idxs0, idxd0, qb0, kb0, vb0, sem_a)

    def pair(i, _):
        fetch(base0 + (2 * i + 1) * C, idxs1, idxd1, qb1, kb1, vb1, sem_b)
        drain(idxs0, idxd0, qb0, kb0, vb0, sem_a)
        compute_scatter(qb0, kb0, vb0, idxd0)
        fetch(base0 + (2 * i + 2) * C, idxs0, idxd0, qb0, kb0, vb0, sem_a)
        drain(idxs1, idxd1, qb1, kb1, vb1, sem_b)
        compute_scatter(qb1, kb1, vb1, idxd1)
        return 0

    lax.fori_loop(0, (CHUNKS - 1) // 2, pair, 0)

    drain(idxs0, idxd0, qb0, kb0, vb0, sem_a)
    compute_scatter(qb0, kb0, vb0, idxd0)


def _sc_body(q0, k0, v0, q1, k1, v1, s0, d0, s1, d1,
             agg0, agg1, den0, den1,
             idxs0, idxd0, idxs1, idxd1,
             qb0, kb0, vb0, qb1, kb1, vb1,
             mb, wb, agg_sh, s_sh, sem_a, sem_b):
    cid = lax.axis_index("c")
    tid = lax.axis_index("s")

    zv = jnp.zeros((LANES,), jnp.float32)

    def zero_mb(i, _):
        r = i // (OUT // LANES)
        cc = i % (OUT // LANES)
        mb[r, pl.ds(cc * LANES, LANES)] = zv
        return 0

    lax.fori_loop(0, C * (OUT // LANES), zero_mb, 0)

    def zero_wb(i, _):
        wb[i, pl.ds(0, LANES)] = zv
        return 0

    lax.fori_loop(0, C, zero_wb, 0)

    def zero_stripe(j, _):
        r0z = tid * ROWS_PER_TILE + j * C
        pltpu.sync_copy(mb, agg_sh.at[pl.ds(r0z, C)])
        pltpu.sync_copy(wb, s_sh.at[pl.ds(r0z, C)])
        return 0

    lax.fori_loop(0, ROWS_PER_TILE // C, zero_stripe, 0)

    plsc.subcore_barrier()

    @pl.when(cid == 0)
    def _():
        _edge_chunks(tid, q0, k0, v0, s0, d0,
                     idxs0, idxd0, idxs1, idxd1,
                     qb0, kb0, vb0, qb1, kb1, vb1,
                     mb, wb, agg_sh, s_sh, sem_a, sem_b)

    @pl.when(cid == 1)
    def _():
        _edge_chunks(tid, q1, k1, v1, s1, d1,
                     idxs0, idxd0, idxs1, idxd1,
                     qb0, kb0, vb0, qb1, kb1, vb1,
                     mb, wb, agg_sh, s_sh, sem_a, sem_b)

    plsc.subcore_barrier()

    r0 = tid * ROWS_PER_TILE

    @pl.when(cid == 0)
    def _():
        pltpu.sync_copy(agg_sh.at[pl.ds(r0, ROWS_PER_TILE)],
                        agg0.at[pl.ds(r0, ROWS_PER_TILE)])
        pltpu.sync_copy(s_sh.at[pl.ds(r0, ROWS_PER_TILE)],
                        den0.at[pl.ds(r0, ROWS_PER_TILE)])

    @pl.when(cid == 1)
    def _():
        pltpu.sync_copy(agg_sh.at[pl.ds(r0, ROWS_PER_TILE)],
                        agg1.at[pl.ds(r0, ROWS_PER_TILE)])
        pltpu.sync_copy(s_sh.at[pl.ds(r0, ROWS_PER_TILE)],
                        den1.at[pl.ds(r0, ROWS_PER_TILE)])


_sc_edge = functools.partial(
    pl.kernel,
    out_type=[
        jax.ShapeDtypeStruct((NPAD, OUT), jnp.float32),
        jax.ShapeDtypeStruct((NPAD, OUT), jnp.float32),
        jax.ShapeDtypeStruct((NPAD, 2 * H), jnp.float32),
        jax.ShapeDtypeStruct((NPAD, 2 * H), jnp.float32),
    ],
    mesh=plsc.VectorSubcoreMesh(core_axis_name="c", subcore_axis_name="s",
                                num_cores=NC, num_subcores=NS),
    compiler_params=pltpu.CompilerParams(needs_layout_passes=False,
                                         use_tc_tiling_on_sc=False),
    scratch_types=[
        pltpu.VMEM((C,), jnp.int32),
        pltpu.VMEM((C,), jnp.int32),
        pltpu.VMEM((C,), jnp.int32),
        pltpu.VMEM((C,), jnp.int32),
        pltpu.VMEM((C, OUT), jnp.float32),
        pltpu.VMEM((C, OUT), jnp.float32),
        pltpu.VMEM((C, OUT), jnp.float32),
        pltpu.VMEM((C, OUT), jnp.float32),
        pltpu.VMEM((C, OUT), jnp.float32),
        pltpu.VMEM((C, OUT), jnp.float32),
        pltpu.VMEM((C, OUT), jnp.float32),
        pltpu.VMEM((C, 2 * H), jnp.float32),
        pltpu.VMEM_SHARED((NPAD, OUT), jnp.float32),
        pltpu.VMEM_SHARED((NPAD, 2 * H), jnp.float32),
        pltpu.SemaphoreType.DMA,
        pltpu.SemaphoreType.DMA,
    ],
)(_sc_body)


def _fold_weights(W_kqv, b_kqv, p_q, a_k, m_v):
    Wk, Wq, Wv = jnp.split(W_kqv, 3, axis=1)
    bk, bq, bv = jnp.split(b_kqv, 3)
    sc = (p_q / jnp.sqrt(jnp.float32(D)))
    Wq2 = (Wq.reshape(IN, H, D) * sc[None, :, None]).reshape(IN, OUT)
    bq2 = (bq.reshape(H, D) * sc[:, None]).reshape(OUT)
    Wk2 = jnp.einsum('ihd,hde->ihe', Wk.reshape(IN, H, D), a_k).reshape(IN, OUT)
    bk2 = jnp.einsum('hd,hde->he', bk.reshape(H, D), a_k).reshape(OUT)
    Wv2 = jnp.einsum('ihd,hde->ihe', Wv.reshape(IN, H, D), m_v).reshape(IN, OUT)
    bv2 = jnp.einsum('hd,hde->he', bv.reshape(H, D), m_v).reshape(OUT)
    W = jnp.concatenate([Wq2, Wk2, Wv2], axis=1)
    b = jnp.concatenate([bq2, bk2, bv2])[None, :]
    return W, b


def kernel(x_paper, x_author, edge_index_writes, edge_index_rev,
           W_kqv_paper, b_kqv_paper, W_kqv_author, b_kqv_author,
           a_writes, m_writes, p_writes, a_rev, m_rev, p_rev,
           W_out_paper, b_out_paper, W_out_author, b_out_author,
           skip_paper, skip_author, ln_gamma, ln_beta):
    W_p, b_p = _fold_weights(W_kqv_paper, b_kqv_paper, p_writes, a_rev, m_rev)
    W_a, b_a = _fold_weights(W_kqv_author, b_kqv_author, p_rev,
                             a_writes, m_writes)

    q_p, kt_p, vt_p = _tc_pre(x_paper, W_p, b_p)
    q_a, kt_a, vt_a = _tc_pre(x_author, W_a, b_a)

    agg_p, agg_a, den_p, den_a = _sc_edge(
        q_p, kt_a, vt_a, q_a, kt_p, vt_p,
        edge_index_writes[0], edge_index_writes[1],
        edge_index_rev[0], edge_index_rev[1])

    al_p = jax.nn.sigmoid(skip_paper).reshape(1, 1)
    al_a = jax.nn.sigmoid(skip_author).reshape(1, 1)
    gamma = ln_gamma[None, :]
    beta = ln_beta[None, :]
    b_out_p = b_out_paper[None, :]
    b_out_a = b_out_author[None, :]

    out_p = _tc_post(agg_p, den_p, x_paper, W_out_paper, b_out_p,
                     al_p, gamma, beta)
    out_a = _tc_post(agg_a, den_a, x_author, W_out_author,
                     b_out_a, al_a, gamma, beta)
    return out_p, out_a


# speedup vs baseline: 4.5018x; 1.4768x over previous
import functools
import jax
import jax.numpy as jnp
from jax import lax
from jax.experimental import pallas as pl
from jax.experimental.pallas import tpu as pltpu
from jax.experimental.pallas import tpu_sc as plsc

H = 8
D = 16
OUT = 128
IN = 128
N = 10000
E = 320000

NC = 2
NS = 16
LANES = 16

C = 32
EPW = E // NS
CHUNKS = EPW // C
NPAD = 10240
ROWS_PER_TILE = NPAD // NS

IB = 25
IBC = IB * C
NBLK = CHUNKS // IB


def _pre_body(x_ref, w_ref, b_ref, q_ref, k_ref, v_ref):
    y = jnp.dot(x_ref[...], w_ref[...], preferred_element_type=jnp.float32)
    y = y + b_ref[...]
    q_ref[...] = y[:, 0:OUT]
    k_ref[...] = y[:, OUT:2 * OUT]
    v_ref[...] = y[:, 2 * OUT:3 * OUT]


def _tc_pre(x, w, b):
    BN = 1000
    grid = (N // BN,)
    return pl.pallas_call(
        _pre_body,
        grid=grid,
        in_specs=[
            pl.BlockSpec((BN, IN), lambda i: (i, 0)),
            pl.BlockSpec((IN, 3 * OUT), lambda i: (0, 0)),
            pl.BlockSpec((1, 3 * OUT), lambda i: (0, 0)),
        ],
        out_specs=[
            pl.BlockSpec((BN, OUT), lambda i: (i, 0)),
            pl.BlockSpec((BN, OUT), lambda i: (i, 0)),
            pl.BlockSpec((BN, OUT), lambda i: (i, 0)),
        ],
        out_shape=[jax.ShapeDtypeStruct((N, OUT), jnp.float32)] * 3,
    )(x, w, b)


def _post_body(agg_ref, s_ref, x_ref, w_ref, b_ref, al_ref, g_ref, be_ref,
               o_ref):
    agg = agg_ref[...]
    den = s_ref[...][:, 0:H] + 1e-16
    bn = agg.shape[0]
    aggn = (agg.reshape(bn, H, D) / den[:, :, None]).reshape(bn, OUT)
    o = jnp.dot(jax.nn.gelu(aggn), w_ref[...],
                preferred_element_type=jnp.float32) + b_ref[...]
    al = al_ref[...]
    r = al * o + (1.0 - al) * x_ref[...]
    r = jnp.maximum(r, 0.0)
    mu = jnp.mean(r, axis=-1, keepdims=True)
    var = jnp.mean((r - mu) ** 2, axis=-1, keepdims=True)
    rn = (r - mu) * lax.rsqrt(var + 1e-5)
    o_ref[...] = g_ref[...] * rn + be_ref[...]


def _tc_post(agg, s, x, w, b, al, gamma, beta):
    BN = 1000
    grid = (N // BN,)
    return pl.pallas_call(
        _post_body,
        grid=grid,
        in_specs=[
            pl.BlockSpec((BN, OUT), lambda i: (i, 0)),
            pl.BlockSpec((BN, 2 * H), lambda i: (i, 0)),
            pl.BlockSpec((BN, IN), lambda i: (i, 0)),
            pl.BlockSpec((OUT, OUT), lambda i: (0, 0)),
            pl.BlockSpec((1, OUT), lambda i: (0, 0)),
            pl.BlockSpec((1, 1), lambda i: (0, 0)),
            pl.BlockSpec((1, OUT), lambda i: (0, 0)),
            pl.BlockSpec((1, OUT), lambda i: (0, 0)),
        ],
        out_specs=pl.BlockSpec((BN, OUT), lambda i: (i, 0)),
        out_shape=jax.ShapeDtypeStruct((N, OUT), jnp.float32),
    )(agg, s, x, w, b, al, gamma, beta)


def _edge_chunks(tid, q_hbm, k_hbm, v_hbm, src_hbm, dst_hbm,
                 isblk, idblk,
                 qb0, kb0, vb0, qb1, kb1, vb1,
                 mb, wb, agg_sh, s_sh, sem_a, sem_b, sem_i):
    base0 = tid * EPW
    lanes = lax.iota(jnp.int32, LANES)

    def islice(c):
        off = (c % (2 * IB)) * C
        return isblk.at[pl.ds(off, C)], idblk.at[pl.ds(off, C)]

    def compute_scatter(qb, kb, vb, idxd):
        def edge(e, _):
            wvec = jnp.zeros((LANES,), jnp.float32)
            for h in range(H):
                prod = qb[e, pl.ds(h * D, D)] * kb[e, pl.ds(h * D, D)]
                sh = jnp.sum(prod)
                wsp = jnp.exp(jnp.full((LANES,), sh, jnp.float32))
                mb[e, pl.ds(h * D, D)] = vb[e, pl.ds(h * D, D)] * wsp
                wvec = jnp.where(lanes == h, wsp, wvec)
            wb[e, pl.ds(0, LANES)] = wvec
            return 0

        lax.fori_loop(0, C, edge, 0)
        pltpu.sync_copy(mb, agg_sh.at[idxd], add=True)
        pltpu.sync_copy(wb, s_sh.at[idxd], add=True)

    def fetch(c, qb, kb, vb, sem):
        si, di = islice(c)
        pltpu.async_copy(q_hbm.at[di], qb, sem)
        pltpu.async_copy(k_hbm.at[si], kb, sem)
        pltpu.async_copy(v_hbm.at[si], vb, sem)

    def drain(c, qb, kb, vb, sem):
        si, di = islice(c)
        pltpu.make_async_copy(q_hbm.at[di], qb, sem).wait()
        pltpu.make_async_copy(k_hbm.at[si], kb, sem).wait()
        pltpu.make_async_copy(v_hbm.at[si], vb, sem).wait()

    def blk_copy(issue, b, half):
        src_v = src_hbm.at[pl.ds(base0 + b * IBC, IBC)]
        dst_v = dst_hbm.at[pl.ds(base0 + b * IBC, IBC)]
        is_v = isblk.at[pl.ds(half * IBC, IBC)]
        id_v = idblk.at[pl.ds(half * IBC, IBC)]
        if issue:
            pltpu.async_copy(src_v, is_v, sem_i)
            pltpu.async_copy(dst_v, id_v, sem_i)
        else:
            pltpu.make_async_copy(src_v, is_v, sem_i).wait()
            pltpu.make_async_copy(dst_v, id_v, sem_i).wait()

    pltpu.sync_copy(src_hbm.at[pl.ds(base0, IBC)], isblk.at[pl.ds(0, IBC)])
    pltpu.sync_copy(dst_hbm.at[pl.ds(base0, IBC)], idblk.at[pl.ds(0, IBC)])
    blk_copy(True, 1, 1)
    fetch(0, qb0, kb0, vb0, sem_a)

    def step(c, _):
        b = c // IB
        last = (c % IB) == (IB - 1)
        more = c + 1 < CHUNKS

        @pl.when(jnp.logical_and(last, more))
        def _():
            blk_copy(False, b + 1, (b + 1) % 2)

        @pl.when(jnp.logical_and(more, (c + 1) % 2 == 0))
        def _():
            fetch(c + 1, qb0, kb0, vb0, sem_a)

        @pl.when(jnp.logical_and(more, (c + 1) % 2 == 1))
        def _():
            fetch(c + 1, qb1, kb1, vb1, sem_b)

        @pl.when(c % 2 == 0)
        def _():
            drain(c, qb0, kb0, vb0, sem_a)
            compute_scatter(qb0, kb0, vb0, islice(c)[1])

        @pl.when(c % 2 == 1)
        def _():
            drain(c, qb1, kb1, vb1, sem_b)
            compute_scatter(qb1, kb1, vb1, islice(c)[1])

        @pl.when(jnp.logical_and(last, b + 2 < NBLK))
        def _():
            blk_copy(True, b + 2, b % 2)

        return 0

    lax.fori_loop(0, CHUNKS, step, 0)


def _sc_body(q0, k0, v0, q1, k1, v1, s0, d0, s1, d1,
             agg0, agg1, den0, den1,
             isblk, idblk,
             qb0, kb0, vb0, qb1, kb1, vb1,
             mb, wb, agg_sh, s_sh, sem_a, sem_b, sem_i):
    cid = lax.axis_index("c")
    tid = lax.axis_index("s")

    zv = jnp.zeros((LANES,), jnp.float32)

    def zero_mb(i, _):
        r = i // (OUT // LANES)
        cc = i % (OUT // LANES)
        mb[r, pl.ds(cc * LANES, LANES)] = zv
        return 0

    lax.fori_loop(0, C * (OUT // LANES), zero_mb, 0)

    def zero_wb(i, _):
        wb[i, pl.ds(0, LANES)] = zv
        return 0

    lax.fori_loop(0, C, zero_wb, 0)

    def zero_stripe(j, _):
        r0z = tid * ROWS_PER_TILE + j * C
        pltpu.sync_copy(mb, agg_sh.at[pl.ds(r0z, C)])
        pltpu.sync_copy(wb, s_sh.at[pl.ds(r0z, C)])
        return 0

    lax.fori_loop(0, ROWS_PER_TILE // C, zero_stripe, 0)

    plsc.subcore_barrier()

    @pl.when(cid == 0)
    def _():
        _edge_chunks(tid, q0, k0, v0, s0, d0,
                     isblk, idblk,
                     qb0, kb0, vb0, qb1, kb1, vb1,
                     mb, wb, agg_sh, s_sh, sem_a, sem_b, sem_i)

    @pl.when(cid == 1)
    def _():
        _edge_chunks(tid, q1, k1, v1, s1, d1,
                     isblk, idblk,
                     qb0, kb0, vb0, qb1, kb1, vb1,
                     mb, wb, agg_sh, s_sh, sem_a, sem_b, sem_i)

    plsc.subcore_barrier()

    r0 = tid * ROWS_PER_TILE

    @pl.when(cid == 0)
    def _():
        pltpu.sync_copy(agg_sh.at[pl.ds(r0, ROWS_PER_TILE)],
                        agg0.at[pl.ds(r0, ROWS_PER_TILE)])
        pltpu.sync_copy(s_sh.at[pl.ds(r0, ROWS_PER_TILE)],
                        den0.at[pl.ds(r0, ROWS_PER_TILE)])

    @pl.when(cid == 1)
    def _():
        pltpu.sync_copy(agg_sh.at[pl.ds(r0, ROWS_PER_TILE)],
                        agg1.at[pl.ds(r0, ROWS_PER_TILE)])
        pltpu.sync_copy(s_sh.at[pl.ds(r0, ROWS_PER_TILE)],
                        den1.at[pl.ds(r0, ROWS_PER_TILE)])


_sc_edge = functools.partial(
    pl.kernel,
    out_type=[
        jax.ShapeDtypeStruct((NPAD, OUT), jnp.float32),
        jax.ShapeDtypeStruct((NPAD, OUT), jnp.float32),
        jax.ShapeDtypeStruct((NPAD, 2 * H), jnp.float32),
        jax.ShapeDtypeStruct((NPAD, 2 * H), jnp.float32),
    ],
    mesh=plsc.VectorSubcoreMesh(core_axis_name="c", subcore_axis_name="s",
                                num_cores=NC, num_subcores=NS),
    compiler_params=pltpu.CompilerParams(needs_layout_passes=False,
                                         use_tc_tiling_on_sc=False),
    scratch_types=[
        pltpu.VMEM((2 * IBC,), jnp.int32),
        pltpu.VMEM((2 * IBC,), jnp.int32),
        pltpu.VMEM((C, OUT), jnp.float32),
        pltpu.VMEM((C, OUT), jnp.float32),
        pltpu.VMEM((C, OUT), jnp.float32),
        pltpu.VMEM((C, OUT), jnp.float32),
        pltpu.VMEM((C, OUT), jnp.float32),
        pltpu.VMEM((C, OUT), jnp.float32),
        pltpu.VMEM((C, OUT), jnp.float32),
        pltpu.VMEM((C, 2 * H), jnp.float32),
        pltpu.VMEM_SHARED((NPAD, OUT), jnp.float32),
        pltpu.VMEM_SHARED((NPAD, 2 * H), jnp.float32),
        pltpu.SemaphoreType.DMA,
        pltpu.SemaphoreType.DMA,
        pltpu.SemaphoreType.DMA,
    ],
)(_sc_body)


def _fold_weights(W_kqv, b_kqv, p_q, a_k, m_v):
    Wk, Wq, Wv = jnp.split(W_kqv, 3, axis=1)
    bk, bq, bv = jnp.split(b_kqv, 3)
    sc = (p_q / jnp.sqrt(jnp.float32(D)))
    Wq2 = (Wq.reshape(IN, H, D) * sc[None, :, None]).reshape(IN, OUT)
    bq2 = (bq.reshape(H, D) * sc[:, None]).reshape(OUT)
    Wk2 = jnp.einsum('ihd,hde->ihe', Wk.reshape(IN, H, D), a_k).reshape(IN, OUT)
    bk2 = jnp.einsum('hd,hde->he', bk.reshape(H, D), a_k).reshape(OUT)
    Wv2 = jnp.einsum('ihd,hde->ihe', Wv.reshape(IN, H, D), m_v).reshape(IN, OUT)
    bv2 = jnp.einsum('hd,hde->he', bv.reshape(H, D), m_v).reshape(OUT)
    W = jnp.concatenate([Wq2, Wk2, Wv2], axis=1)
    b = jnp.concatenate([bq2, bk2, bv2])[None, :]
    return W, b


def kernel(x_paper, x_author, edge_index_writes, edge_index_rev,
           W_kqv_paper, b_kqv_paper, W_kqv_author, b_kqv_author,
           a_writes, m_writes, p_writes, a_rev, m_rev, p_rev,
           W_out_paper, b_out_paper, W_out_author, b_out_author,
           skip_paper, skip_author, ln_gamma, ln_beta):
    W_p, b_p = _fold_weights(W_kqv_paper, b_kqv_paper, p_writes, a_rev, m_rev)
    W_a, b_a = _fold_weights(W_kqv_author, b_kqv_author, p_rev,
                             a_writes, m_writes)

    q_p, kt_p, vt_p = _tc_pre(x_paper, W_p, b_p)
    q_a, kt_a, vt_a = _tc_pre(x_author, W_a, b_a)

    agg_p, agg_a, den_p, den_a = _sc_edge(
        q_p, kt_a, vt_a, q_a, kt_p, vt_p,
        edge_index_writes[0], edge_index_writes[1],
        edge_index_rev[0], edge_index_rev[1])

    al_p = jax.nn.sigmoid(skip_paper).reshape(1, 1)
    al_a = jax.nn.sigmoid(skip_author).reshape(1, 1)
    gamma = ln_gamma[None, :]
    beta = ln_beta[None, :]
    b_out_p = b_out_paper[None, :]
    b_out_a = b_out_author[None, :]

    out_p = _tc_post(agg_p, den_p, x_paper, W_out_paper, b_out_p,
                     al_p, gamma, beta)
    out_a = _tc_post(agg_a, den_a, x_author, W_out_author,
                     b_out_a, al_a, gamma, beta)
    return out_p, out_a


# speedup vs baseline: 4.5236x; 1.0048x over previous
import functools
import jax
import jax.numpy as jnp
from jax import lax
from jax.experimental import pallas as pl
from jax.experimental.pallas import tpu as pltpu
from jax.experimental.pallas import tpu_sc as plsc

H = 8
D = 16
OUT = 128
IN = 128
N = 10000
E = 320000

NC = 2
NS = 16
LANES = 16

C = 32
EPW = E // NS
CHUNKS = EPW // C
NPAD = 10240
ROWS_PER_TILE = NPAD // NS

IB = 25
IBC = IB * C
NBLK = CHUNKS // IB


def _pre_body(x_ref, w_ref, b_ref, q_ref, k_ref, v_ref):
    y = jnp.dot(x_ref[...], w_ref[...], preferred_element_type=jnp.float32)
    y = y + b_ref[...]
    q_ref[...] = y[:, 0:OUT]
    k_ref[...] = y[:, OUT:2 * OUT]
    v_ref[...] = y[:, 2 * OUT:3 * OUT]


def _tc_pre(x, w, b):
    BN = 1000
    grid = (N // BN,)
    return pl.pallas_call(
        _pre_body,
        grid=grid,
        in_specs=[
            pl.BlockSpec((BN, IN), lambda i: (i, 0)),
            pl.BlockSpec((IN, 3 * OUT), lambda i: (0, 0)),
            pl.BlockSpec((1, 3 * OUT), lambda i: (0, 0)),
        ],
        out_specs=[
            pl.BlockSpec((BN, OUT), lambda i: (i, 0)),
            pl.BlockSpec((BN, OUT), lambda i: (i, 0)),
            pl.BlockSpec((BN, OUT), lambda i: (i, 0)),
        ],
        out_shape=[jax.ShapeDtypeStruct((N, OUT), jnp.float32)] * 3,
    )(x, w, b)


def _post_body(agg_ref, s_ref, x_ref, w_ref, b_ref, al_ref, g_ref, be_ref,
               o_ref):
    agg = agg_ref[...]
    den = s_ref[...][:, 0:H] + 1e-16
    bn = agg.shape[0]
    aggn = (agg.reshape(bn, H, D) / den[:, :, None]).reshape(bn, OUT)
    o = jnp.dot(jax.nn.gelu(aggn), w_ref[...],
                preferred_element_type=jnp.float32) + b_ref[...]
    al = al_ref[...]
    r = al * o + (1.0 - al) * x_ref[...]
    r = jnp.maximum(r, 0.0)
    mu = jnp.mean(r, axis=-1, keepdims=True)
    var = jnp.mean((r - mu) ** 2, axis=-1, keepdims=True)
    rn = (r - mu) * lax.rsqrt(var + 1e-5)
    o_ref[...] = g_ref[...] * rn + be_ref[...]


def _tc_post(agg, s, x, w, b, al, gamma, beta):
    BN = 1000
    grid = (N // BN,)
    return pl.pallas_call(
        _post_body,
        grid=grid,
        in_specs=[
            pl.BlockSpec((BN, OUT), lambda i: (i, 0)),
            pl.BlockSpec((BN, 2 * H), lambda i: (i, 0)),
            pl.BlockSpec((BN, IN), lambda i: (i, 0)),
            pl.BlockSpec((OUT, OUT), lambda i: (0, 0)),
            pl.BlockSpec((1, OUT), lambda i: (0, 0)),
            pl.BlockSpec((1, 1), lambda i: (0, 0)),
            pl.BlockSpec((1, OUT), lambda i: (0, 0)),
            pl.BlockSpec((1, OUT), lambda i: (0, 0)),
        ],
        out_specs=pl.BlockSpec((BN, OUT), lambda i: (i, 0)),
        out_shape=jax.ShapeDtypeStruct((N, OUT), jnp.float32),
    )(agg, s, x, w, b, al, gamma, beta)


def _edge_chunks(tid, q_hbm, k_hbm, v_hbm, src_hbm, dst_hbm,
                 isblk, idblk,
                 qb0, kb0, vb0, qb1, kb1, vb1,
                 mb, wb, agg_sh, s_sh, sem_a, sem_b, sem_i):
    base0 = tid * EPW
    lanes = lax.iota(jnp.int32, LANES)

    def islice(c):
        off = (c % (2 * IB)) * C
        return isblk.at[pl.ds(off, C)], idblk.at[pl.ds(off, C)]

    def compute_scatter(qb, kb, vb, idxd):
        def edge(e2, _):
            for e in (2 * e2, 2 * e2 + 1):
                wvec = jnp.zeros((LANES,), jnp.float32)
                for h in range(H):
                    prod = qb[e, pl.ds(h * D, D)] * kb[e, pl.ds(h * D, D)]
                    sh = jnp.sum(prod)
                    wsp = jnp.exp(jnp.full((LANES,), sh, jnp.float32))
                    mb[e, pl.ds(h * D, D)] = vb[e, pl.ds(h * D, D)] * wsp
                    wvec = jnp.where(lanes == h, wsp, wvec)
                wb[e, pl.ds(0, LANES)] = wvec
            return 0

        lax.fori_loop(0, C // 2, edge, 0)
        pltpu.sync_copy(mb, agg_sh.at[idxd], add=True)
        pltpu.sync_copy(wb, s_sh.at[idxd], add=True)

    def fetch(c, qb, kb, vb, sem):
        si, di = islice(c)
        pltpu.async_copy(q_hbm.at[di], qb, sem)
        pltpu.async_copy(k_hbm.at[si], kb, sem)
        pltpu.async_copy(v_hbm.at[si], vb, sem)

    def drain(c, qb, kb, vb, sem):
        si, di = islice(c)
        pltpu.make_async_copy(q_hbm.at[di], qb, sem).wait()
        pltpu.make_async_copy(k_hbm.at[si], kb, sem).wait()
        pltpu.make_async_copy(v_hbm.at[si], vb, sem).wait()

    def blk_copy(issue, b, half):
        src_v = src_hbm.at[pl.ds(base0 + b * IBC, IBC)]
        dst_v = dst_hbm.at[pl.ds(base0 + b * IBC, IBC)]
        is_v = isblk.at[pl.ds(half * IBC, IBC)]
        id_v = idblk.at[pl.ds(half * IBC, IBC)]
        if issue:
            pltpu.async_copy(src_v, is_v, sem_i)
            pltpu.async_copy(dst_v, id_v, sem_i)
        else:
            pltpu.make_async_copy(src_v, is_v, sem_i).wait()
            pltpu.make_async_copy(dst_v, id_v, sem_i).wait()

    pltpu.sync_copy(src_hbm.at[pl.ds(base0, IBC)], isblk.at[pl.ds(0, IBC)])
    pltpu.sync_copy(dst_hbm.at[pl.ds(base0, IBC)], idblk.at[pl.ds(0, IBC)])
    blk_copy(True, 1, 1)
    fetch(0, qb0, kb0, vb0, sem_a)

    def step(c, _):
        b = c // IB
        last = (c % IB) == (IB - 1)
        more = c + 1 < CHUNKS

        @pl.when(jnp.logical_and(last, more))
        def _():
            blk_copy(False, b + 1, (b + 1) % 2)

        @pl.when(jnp.logical_and(more, (c + 1) % 2 == 0))
        def _():
            fetch(c + 1, qb0, kb0, vb0, sem_a)

        @pl.when(jnp.logical_and(more, (c + 1) % 2 == 1))
        def _():
            fetch(c + 1, qb1, kb1, vb1, sem_b)

        @pl.when(c % 2 == 0)
        def _():
            drain(c, qb0, kb0, vb0, sem_a)
            compute_scatter(qb0, kb0, vb0, islice(c)[1])

        @pl.when(c % 2 == 1)
        def _():
            drain(c, qb1, kb1, vb1, sem_b)
            compute_scatter(qb1, kb1, vb1, islice(c)[1])

        @pl.when(jnp.logical_and(last, b + 2 < NBLK))
        def _():
            blk_copy(True, b + 2, b % 2)

        return 0

    lax.fori_loop(0, CHUNKS, step, 0)


def _sc_body(q0, k0, v0, q1, k1, v1, s0, d0, s1, d1,
             agg0, agg1, den0, den1,
             isblk, idblk,
             qb0, kb0, vb0, qb1, kb1, vb1,
             mb, wb, agg_sh, s_sh, sem_a, sem_b, sem_i):
    cid = lax.axis_index("c")
    tid = lax.axis_index("s")

    zv = jnp.zeros((LANES,), jnp.float32)

    def zero_mb(i, _):
        r = i // (OUT // LANES)
        cc = i % (OUT // LANES)
        mb[r, pl.ds(cc * LANES, LANES)] = zv
        return 0

    lax.fori_loop(0, C * (OUT // LANES), zero_mb, 0)

    def zero_wb(i, _):
        wb[i, pl.ds(0, LANES)] = zv
        return 0

    lax.fori_loop(0, C, zero_wb, 0)

    def zero_stripe(j, _):
        r0z = tid * ROWS_PER_TILE + j * C
        pltpu.sync_copy(mb, agg_sh.at[pl.ds(r0z, C)])
        pltpu.sync_copy(wb, s_sh.at[pl.ds(r0z, C)])
        return 0

    lax.fori_loop(0, ROWS_PER_TILE // C, zero_stripe, 0)

    plsc.subcore_barrier()

    @pl.when(cid == 0)
    def _():
        _edge_chunks(tid, q0, k0, v0, s0, d0,
                     isblk, idblk,
                     qb0, kb0, vb0, qb1, kb1, vb1,
                     mb, wb, agg_sh, s_sh, sem_a, sem_b, sem_i)

    @pl.when(cid == 1)
    def _():
        _edge_chunks(tid, q1, k1, v1, s1, d1,
                     isblk, idblk,
                     qb0, kb0, vb0, qb1, kb1, vb1,
                     mb, wb, agg_sh, s_sh, sem_a, sem_b, sem_i)

    plsc.subcore_barrier()

    r0 = tid * ROWS_PER_TILE

    @pl.when(cid == 0)
    def _():
        pltpu.sync_copy(agg_sh.at[pl.ds(r0, ROWS_PER_TILE)],
                        agg0.at[pl.ds(r0, ROWS_PER_TILE)])
        pltpu.sync_copy(s_sh.at[pl.ds(r0, ROWS_PER_TILE)],
                        den0.at[pl.ds(r0, ROWS_PER_TILE)])

    @pl.when(cid == 1)
    def _():
        pltpu.sync_copy(agg_sh.at[pl.ds(r0, ROWS_PER_TILE)],
                        agg1.at[pl.ds(r0, ROWS_PER_TILE)])
        pltpu.sync_copy(s_sh.at[pl.ds(r0, ROWS_PER_TILE)],
                        den1.at[pl.ds(r0, ROWS_PER_TILE)])


_sc_edge = functools.partial(
    pl.kernel,
    out_type=[
        jax.ShapeDtypeStruct((NPAD, OUT), jnp.float32),
        jax.ShapeDtypeStruct((NPAD, OUT), jnp.float32),
        jax.ShapeDtypeStruct((NPAD, 2 * H), jnp.float32),
        jax.ShapeDtypeStruct((NPAD, 2 * H), jnp.float32),
    ],
    mesh=plsc.VectorSubcoreMesh(core_axis_name="c", subcore_axis_name="s",
                                num_cores=NC, num_subcores=NS),
    compiler_params=pltpu.CompilerParams(needs_layout_passes=False,
                                         use_tc_tiling_on_sc=False),
    scratch_types=[
        pltpu.VMEM((2 * IBC,), jnp.int32),
        pltpu.VMEM((2 * IBC,), jnp.int32),
        pltpu.VMEM((C, OUT), jnp.float32),
        pltpu.VMEM((C, OUT), jnp.float32),
        pltpu.VMEM((C, OUT), jnp.float32),
        pltpu.VMEM((C, OUT), jnp.float32),
        pltpu.VMEM((C, OUT), jnp.float32),
        pltpu.VMEM((C, OUT), jnp.float32),
        pltpu.VMEM((C, OUT), jnp.float32),
        pltpu.VMEM((C, 2 * H), jnp.float32),
        pltpu.VMEM_SHARED((NPAD, OUT), jnp.float32),
        pltpu.VMEM_SHARED((NPAD, 2 * H), jnp.float32),
        pltpu.SemaphoreType.DMA,
        pltpu.SemaphoreType.DMA,
        pltpu.SemaphoreType.DMA,
    ],
)(_sc_body)


def _fold_weights(W_kqv, b_kqv, p_q, a_k, m_v):
    Wk, Wq, Wv = jnp.split(W_kqv, 3, axis=1)
    bk, bq, bv = jnp.split(b_kqv, 3)
    sc = (p_q / jnp.sqrt(jnp.float32(D)))
    Wq2 = (Wq.reshape(IN, H, D) * sc[None, :, None]).reshape(IN, OUT)
    bq2 = (bq.reshape(H, D) * sc[:, None]).reshape(OUT)
    Wk2 = jnp.einsum('ihd,hde->ihe', Wk.reshape(IN, H, D), a_k).reshape(IN, OUT)
    bk2 = jnp.einsum('hd,hde->he', bk.reshape(H, D), a_k).reshape(OUT)
    Wv2 = jnp.einsum('ihd,hde->ihe', Wv.reshape(IN, H, D), m_v).reshape(IN, OUT)
    bv2 = jnp.einsum('hd,hde->he', bv.reshape(H, D), m_v).reshape(OUT)
    W = jnp.concatenate([Wq2, Wk2, Wv2], axis=1)
    b = jnp.concatenate([bq2, bk2, bv2])[None, :]
    return W, b


def kernel(x_paper, x_author, edge_index_writes, edge_index_rev,
           W_kqv_paper, b_kqv_paper, W_kqv_author, b_kqv_author,
           a_writes, m_writes, p_writes, a_rev, m_rev, p_rev,
           W_out_paper, b_out_paper, W_out_author, b_out_author,
           skip_paper, skip_author, ln_gamma, ln_beta):
    W_p, b_p = _fold_weights(W_kqv_paper, b_kqv_paper, p_writes, a_rev, m_rev)
    W_a, b_a = _fold_weights(W_kqv_author, b_kqv_author, p_rev,
                             a_writes, m_writes)

    q_p, kt_p, vt_p = _tc_pre(x_paper, W_p, b_p)
    q_a, kt_a, vt_a = _tc_pre(x_author, W_a, b_a)

    agg_p, agg_a, den_p, den_a = _sc_edge(
        q_p, kt_a, vt_a, q_a, kt_p, vt_p,
        edge_index_writes[0], edge_index_writes[1],
        edge_index_rev[0], edge_index_rev[1])

    al_p = jax.nn.sigmoid(skip_paper).reshape(1, 1)
    al_a = jax.nn.sigmoid(skip_author).reshape(1, 1)
    gamma = ln_gamma[None, :]
    beta = ln_beta[None, :]
    b_out_p = b_out_paper[None, :]
    b_out_a = b_out_author[None, :]

    out_p = _tc_post(agg_p, den_p, x_paper, W_out_paper, b_out_p,
                     al_p, gamma, beta)
    out_a = _tc_post(agg_a, den_a, x_author, W_out_author,
                     b_out_a, al_a, gamma, beta)
    return out_p, out_a
